# Initial kernel scaffold; baseline (speedup 1.0000x reference)
import functools

import jax
import jax.numpy as jnp
from jax import lax
from jax.experimental import pallas as pl
from jax.experimental.pallas import tpu as pltpu
from jax.experimental.pallas import tpu_sc as plsc

NUM_ENTITIES = 1000000
HIDDEN_DIM = 64
BATCH = 16384
HIST = 50

NC = 2
NS = 16
NW = NC * NS
CH = 128
CPG = 4
GROW = CH * CPG

TOTAL = BATCH * HIST
PER_W = TOTAL // NW
NGRP = PER_W // GROW
NCHUNK = PER_W // CH


def _make_gather():
    mesh = plsc.VectorSubcoreMesh(core_axis_name="c", subcore_axis_name="s")

    @functools.partial(
        pl.kernel,
        mesh=mesh,
        out_type=jax.ShapeDtypeStruct((TOTAL, HIDDEN_DIM), jnp.float32),
        scratch_types=[
            pltpu.VMEM((NCHUNK, CH), jnp.int32),
            pltpu.VMEM((GROW, HIDDEN_DIM), jnp.float32),
            pltpu.VMEM((GROW, HIDDEN_DIM), jnp.float32),
            pltpu.SemaphoreType.DMA,
            pltpu.SemaphoreType.DMA,
            pltpu.SemaphoreType.DMA,
            pltpu.SemaphoreType.DMA,
        ],
    )
    def gather_kernel(idx_hbm, table_hbm, out_hbm, idx_v, bank0, bank1,
                      gsem0, gsem1, wsem0, wsem1):
        wid = lax.axis_index("c") * NS + lax.axis_index("s")
        base = wid * PER_W
        banks = (bank0, bank1)
        gsems = (gsem0, gsem1)
        wsems = (wsem0, wsem1)

        pltpu.sync_copy(idx_hbm.at[wid], idx_v)

        def gather_desc(g, k, b):
            return pltpu.make_async_copy(
                table_hbm.at[idx_v.at[g * CPG + k]],
                banks[b].at[pl.ds(k * CH, CH)],
                gsems[b],
            )

        def write_desc(g, b):
            return pltpu.make_async_copy(
                banks[b],
                out_hbm.at[pl.ds(base + g * GROW, GROW)],
                wsems[b],
            )

        def start_gathers(g, b):
            for k in range(CPG):
                gather_desc(g, k, b).start()

        def wait_gathers(g, b):
            for k in range(CPG):
                gather_desc(g, k, b).wait()

        start_gathers(0, 0)
        start_gathers(1, 1)
        wait_gathers(0, 0)
        write_desc(0, 0).start()

        def do_group(g, b):
            write_desc(g - 2, b).wait()
            start_gathers(g, b)
            wait_gathers(g - 1, 1 - b)
            write_desc(g - 1, 1 - b).start()

        def body(t, carry):
            do_group(2 * t, 0)
            do_group(2 * t + 1, 1)
            return carry

        lax.fori_loop(1, NGRP // 2, body, 0)

        wait_gathers(NGRP - 1, 1)
        write_desc(NGRP - 1, 1).start()
        write_desc(NGRP - 2, 0).wait()
        write_desc(NGRP - 1, 1).wait()

    return gather_kernel


_gather = _make_gather()


def kernel(entity_ids, table):
    ids = jnp.asarray(entity_ids, jnp.int32).reshape(NW, NCHUNK, CH)
    out = _gather(ids, table)
    return out.reshape(BATCH, HIST, HIDDEN_DIM)


# speedup vs baseline: 1.8727x; 1.8727x over previous
import functools

import jax
import jax.numpy as jnp
from jax import lax
from jax.experimental import pallas as pl
from jax.experimental.pallas import tpu as pltpu
from jax.experimental.pallas import tpu_sc as plsc

NUM_ENTITIES = 1000000
HIDDEN_DIM = 64
BATCH = 16384
HIST = 50

NC = 2
NS = 16
NW = NC * NS
CH = 128
CPG = 4
GROW = CH * CPG

TOTAL = BATCH * HIST
PER_W = TOTAL // NW
NGRP = PER_W // GROW
NCHUNK = PER_W // CH


def _make_gather():
    mesh = plsc.VectorSubcoreMesh(core_axis_name="c", subcore_axis_name="s")

    @functools.partial(
        pl.kernel,
        mesh=mesh,
        out_type=jax.ShapeDtypeStruct((TOTAL, HIDDEN_DIM), jnp.float32),
        compiler_params=pltpu.CompilerParams(use_tc_tiling_on_sc=False),
        scratch_types=[
            pltpu.VMEM((NCHUNK, CH), jnp.int32),
            pltpu.VMEM((GROW, HIDDEN_DIM), jnp.float32),
            pltpu.VMEM((GROW, HIDDEN_DIM), jnp.float32),
            pltpu.SemaphoreType.DMA,
            pltpu.SemaphoreType.DMA,
            pltpu.SemaphoreType.DMA,
            pltpu.SemaphoreType.DMA,
        ],
    )
    def gather_kernel(idx_hbm, table_hbm, out_hbm, idx_v, bank0, bank1,
                      gsem0, gsem1, wsem0, wsem1):
        wid = lax.axis_index("c") * NS + lax.axis_index("s")
        base = wid * PER_W
        banks = (bank0, bank1)
        gsems = (gsem0, gsem1)
        wsems = (wsem0, wsem1)

        pltpu.sync_copy(idx_hbm.at[wid], idx_v)

        def gather_desc(g, k, b):
            return pltpu.make_async_copy(
                table_hbm.at[idx_v.at[g * CPG + k]],
                banks[b].at[pl.ds(k * CH, CH)],
                gsems[b],
            )

        def write_desc(g, b):
            return pltpu.make_async_copy(
                banks[b],
                out_hbm.at[pl.ds(base + g * GROW, GROW)],
                wsems[b],
            )

        def start_gathers(g, b):
            for k in range(CPG):
                gather_desc(g, k, b).start()

        def wait_gathers(g, b):
            for k in range(CPG):
                gather_desc(g, k, b).wait()

        start_gathers(0, 0)
        start_gathers(1, 1)
        wait_gathers(0, 0)
        write_desc(0, 0).start()

        def do_group(g, b):
            write_desc(g - 2, b).wait()
            start_gathers(g, b)
            wait_gathers(g - 1, 1 - b)
            write_desc(g - 1, 1 - b).start()

        def body(t, carry):
            do_group(2 * t, 0)
            do_group(2 * t + 1, 1)
            return carry

        lax.fori_loop(1, NGRP // 2, body, 0)

        wait_gathers(NGRP - 1, 1)
        write_desc(NGRP - 1, 1).start()
        write_desc(NGRP - 2, 0).wait()
        write_desc(NGRP - 1, 1).wait()

    return gather_kernel


_gather = _make_gather()


def kernel(entity_ids, table):
    ids = jnp.asarray(entity_ids, jnp.int32).reshape(NW, NCHUNK, CH)
    out = _gather(ids, table)
    return out.reshape(BATCH, HIST, HIDDEN_DIM)


# speedup vs baseline: 2.1651x; 1.1562x over previous
import functools

import jax
import jax.numpy as jnp
from jax import lax
from jax.experimental import pallas as pl
from jax.experimental.pallas import tpu as pltpu
from jax.experimental.pallas import tpu_sc as plsc

NUM_ENTITIES = 1000000
HIDDEN_DIM = 64
BATCH = 16384
HIST = 50

NC = 2
NS = 16
NW = NC * NS
CH = 128
CPG = 4
GROW = CH * CPG

TOTAL = BATCH * HIST
PER_W = TOTAL // NW
NGRP = PER_W // GROW
NCHUNK = PER_W // CH


_T1_BLK_IN = 1024
_T1_BLK_OUT = _T1_BLK_IN // 2


def _t1_body(in_ref, out_ref):
    y = in_ref[...].T
    out_ref[...] = jnp.concatenate(
        [y[:_T1_BLK_OUT], y[_T1_BLK_OUT:]], axis=1)


_T1_GRID = (NUM_ENTITIES + _T1_BLK_IN - 1) // _T1_BLK_IN
N_PAD = _T1_GRID * _T1_BLK_IN


def _t1(table_t):
    return pl.pallas_call(
        _t1_body,
        grid=(_T1_GRID,),
        in_specs=[pl.BlockSpec((HIDDEN_DIM, _T1_BLK_IN), lambda g: (0, g))],
        out_specs=pl.BlockSpec((_T1_BLK_OUT, 2 * HIDDEN_DIM), lambda g: (g, 0)),
        out_shape=jax.ShapeDtypeStruct(
            (N_PAD // 2, 2 * HIDDEN_DIM), jnp.float32),
    )(table_t)


_T2_BLK = 128


def _t2_body(in_ref, out_ref):
    x = in_ref[...]
    x = x.reshape(_T2_BLK, 25, 128)
    for q in range(25):
        out_ref[q * 128:(q + 1) * 128, :] = x[:, q, :].T


def _t2(lin):
    return pl.pallas_call(
        _t2_body,
        grid=(BATCH // _T2_BLK,),
        in_specs=[pl.BlockSpec((25 * _T2_BLK, 128), lambda g: (g, 0))],
        out_specs=pl.BlockSpec((HIST * HIDDEN_DIM, _T2_BLK), lambda g: (0, g)),
        out_shape=jax.ShapeDtypeStruct((HIST * HIDDEN_DIM, BATCH), jnp.float32),
    )(lin)


def _make_gather():
    mesh = plsc.VectorSubcoreMesh(core_axis_name="c", subcore_axis_name="s")

    @functools.partial(
        pl.kernel,
        mesh=mesh,
        out_type=jax.ShapeDtypeStruct((TOTAL, HIDDEN_DIM), jnp.float32),
        compiler_params=pltpu.CompilerParams(use_tc_tiling_on_sc=False),
        scratch_types=[
            pltpu.VMEM((NCHUNK, CH), jnp.int32),
            pltpu.VMEM((GROW, HIDDEN_DIM), jnp.float32),
            pltpu.VMEM((GROW, HIDDEN_DIM), jnp.float32),
            pltpu.SemaphoreType.DMA,
            pltpu.SemaphoreType.DMA,
            pltpu.SemaphoreType.DMA,
            pltpu.SemaphoreType.DMA,
        ],
    )
    def gather_kernel(idx_hbm, table_hbm, out_hbm, idx_v, bank0, bank1,
                      gsem0, gsem1, wsem0, wsem1):
        wid = lax.axis_index("c") * NS + lax.axis_index("s")
        base = wid * PER_W
        banks = (bank0, bank1)
        gsems = (gsem0, gsem1)
        wsems = (wsem0, wsem1)

        pltpu.sync_copy(idx_hbm.at[wid], idx_v)

        def gather_desc(g, k, b):
            return pltpu.make_async_copy(
                table_hbm.at[idx_v.at[g * CPG + k]],
                banks[b].at[pl.ds(k * CH, CH)],
                gsems[b],
            )

        def write_desc(g, b):
            return pltpu.make_async_copy(
                banks[b],
                out_hbm.at[pl.ds(base + g * GROW, GROW)],
                wsems[b],
            )

        def start_gathers(g, b):
            for k in range(CPG):
                gather_desc(g, k, b).start()

        def wait_gathers(g, b):
            for k in range(CPG):
                gather_desc(g, k, b).wait()

        start_gathers(0, 0)
        start_gathers(1, 1)
        wait_gathers(0, 0)
        write_desc(0, 0).start()

        def do_group(g, b):
            write_desc(g - 2, b).wait()
            start_gathers(g, b)
            wait_gathers(g - 1, 1 - b)
            write_desc(g - 1, 1 - b).start()

        def body(t, carry):
            do_group(2 * t, 0)
            do_group(2 * t + 1, 1)
            return carry

        lax.fori_loop(1, NGRP // 2, body, 0)

        wait_gathers(NGRP - 1, 1)
        write_desc(NGRP - 1, 1).start()
        write_desc(NGRP - 2, 0).wait()
        write_desc(NGRP - 1, 1).wait()

    return gather_kernel


_gather = _make_gather()


def kernel(entity_ids, table):
    i = jnp.asarray(entity_ids, jnp.int32)
    r = (i & ~1023) | ((i & 511) << 1) | ((i >> 9) & 1)
    ids = r.reshape(NW, NCHUNK, CH)
    table_rm = _t1(table.T).reshape(N_PAD, HIDDEN_DIM)
    out_lin = _gather(ids, table_rm)
    o = _t2(out_lin.reshape(TOTAL // 2, 2 * HIDDEN_DIM))
    return o.reshape(HIST, HIDDEN_DIM, BATCH).transpose(2, 0, 1)


# speedup vs baseline: 3.0253x; 1.3973x over previous
import functools

import jax
import jax.numpy as jnp
from jax import lax
from jax.experimental import pallas as pl
from jax.experimental.pallas import tpu as pltpu
from jax.experimental.pallas import tpu_sc as plsc

NUM_ENTITIES = 1000000
HIDDEN_DIM = 64
BATCH = 16384
HIST = 50

NC = 2
NS = 16
NW = NC * NS
CH = 128
CPG = 4
GROW = CH * CPG

TOTAL = BATCH * HIST
PER_W = TOTAL // NW
NGRP = PER_W // GROW
NCHUNK = PER_W // CH


_T1_W = 1024
_T1_SPAN = 2 * _T1_W
_T1_GRID = (NUM_ENTITIES + _T1_SPAN - 1) // _T1_SPAN
_T1_LASTBLK = (NUM_ENTITIES - 1) // _T1_W
N_PAD = _T1_GRID * _T1_SPAN


def _t1_body(a_ref, b_ref, out_ref):
    z = jnp.concatenate([a_ref[...], b_ref[...]], axis=0)
    out_ref[...] = z.T


def _t1(table_t):
    return pl.pallas_call(
        _t1_body,
        grid=(_T1_GRID,),
        in_specs=[
            pl.BlockSpec((HIDDEN_DIM, _T1_W), lambda g: (0, 2 * g)),
            pl.BlockSpec(
                (HIDDEN_DIM, _T1_W),
                lambda g: (0, jnp.minimum(2 * g + 1, _T1_LASTBLK))),
        ],
        out_specs=pl.BlockSpec((_T1_W, 2 * HIDDEN_DIM), lambda g: (g, 0)),
        out_shape=jax.ShapeDtypeStruct(
            (N_PAD // 2, 2 * HIDDEN_DIM), jnp.float32),
    )(table_t, table_t)


_T2_BLK = 128


def _t2_body(in_ref, out_ref):
    x = in_ref[...]
    x = x.reshape(_T2_BLK, 25, 128)
    for q in range(25):
        out_ref[q * 128:(q + 1) * 128, :] = x[:, q, :].T


def _t2(lin):
    return pl.pallas_call(
        _t2_body,
        grid=(BATCH // _T2_BLK,),
        in_specs=[pl.BlockSpec((25 * _T2_BLK, 128), lambda g: (g, 0))],
        out_specs=pl.BlockSpec((HIST * HIDDEN_DIM, _T2_BLK), lambda g: (0, g)),
        out_shape=jax.ShapeDtypeStruct((HIST * HIDDEN_DIM, BATCH), jnp.float32),
    )(lin)


def _make_gather():
    mesh = plsc.VectorSubcoreMesh(core_axis_name="c", subcore_axis_name="s")

    @functools.partial(
        pl.kernel,
        mesh=mesh,
        out_type=jax.ShapeDtypeStruct((TOTAL, HIDDEN_DIM), jnp.float32),
        compiler_params=pltpu.CompilerParams(use_tc_tiling_on_sc=False),
        scratch_types=[
            pltpu.VMEM((NCHUNK, CH), jnp.int32),
            pltpu.VMEM((GROW, HIDDEN_DIM), jnp.float32),
            pltpu.VMEM((GROW, HIDDEN_DIM), jnp.float32),
            pltpu.SemaphoreType.DMA,
            pltpu.SemaphoreType.DMA,
            pltpu.SemaphoreType.DMA,
            pltpu.SemaphoreType.DMA,
        ],
    )
    def gather_kernel(idx_hbm, table_hbm, out_hbm, idx_v, bank0, bank1,
                      gsem0, gsem1, wsem0, wsem1):
        wid = lax.axis_index("c") * NS + lax.axis_index("s")
        base = wid * PER_W
        banks = (bank0, bank1)
        gsems = (gsem0, gsem1)
        wsems = (wsem0, wsem1)

        pltpu.sync_copy(idx_hbm.at[wid], idx_v)

        def gather_desc(g, k, b):
            return pltpu.make_async_copy(
                table_hbm.at[idx_v.at[g * CPG + k]],
                banks[b].at[pl.ds(k * CH, CH)],
                gsems[b],
            )

        def write_desc(g, b):
            return pltpu.make_async_copy(
                banks[b],
                out_hbm.at[pl.ds(base + g * GROW, GROW)],
                wsems[b],
            )

        def start_gathers(g, b):
            for k in range(CPG):
                gather_desc(g, k, b).start()

        def wait_gathers(g, b):
            for k in range(CPG):
                gather_desc(g, k, b).wait()

        start_gathers(0, 0)
        start_gathers(1, 1)
        wait_gathers(0, 0)
        write_desc(0, 0).start()

        def do_group(g, b):
            write_desc(g - 2, b).wait()
            start_gathers(g, b)
            wait_gathers(g - 1, 1 - b)
            write_desc(g - 1, 1 - b).start()

        def body(t, carry):
            do_group(2 * t, 0)
            do_group(2 * t + 1, 1)
            return carry

        lax.fori_loop(1, NGRP // 2, body, 0)

        wait_gathers(NGRP - 1, 1)
        write_desc(NGRP - 1, 1).start()
        write_desc(NGRP - 2, 0).wait()
        write_desc(NGRP - 1, 1).wait()

    return gather_kernel


_gather = _make_gather()


def kernel(entity_ids, table):
    i = jnp.asarray(entity_ids, jnp.int32)
    r = (i & ~2047) | ((i & 1023) << 1) | ((i >> 10) & 1)
    ids = r.reshape(NW, NCHUNK, CH)
    table_rm = _t1(table.T).reshape(N_PAD, HIDDEN_DIM)
    out_lin = _gather(ids, table_rm)
    o = _t2(out_lin.reshape(TOTAL // 2, 2 * HIDDEN_DIM))
    return o.reshape(HIST, HIDDEN_DIM, BATCH).transpose(2, 0, 1)


# speedup vs baseline: 3.7256x; 1.2315x over previous
import functools

import jax
import jax.numpy as jnp
from jax import lax
from jax.experimental import pallas as pl
from jax.experimental.pallas import tpu as pltpu
from jax.experimental.pallas import tpu_sc as plsc

NUM_ENTITIES = 1000000
HIDDEN_DIM = 64
BATCH = 16384
HIST = 50

NC = 2
NS = 16
NW = NC * NS
CH = 128
CPG = 4
GROW = CH * CPG

TOTAL = BATCH * HIST
PER_W = TOTAL // NW
NGRP = PER_W // GROW
NCHUNK = PER_W // CH


_T1_W = 2048
_T1_SPAN = 2 * _T1_W
_T1_SHIFT = _T1_W.bit_length() - 1
_T1_GRID = (NUM_ENTITIES + _T1_SPAN - 1) // _T1_SPAN
_T1_LASTBLK = (NUM_ENTITIES - 1) // _T1_W
N_PAD = _T1_GRID * _T1_SPAN


def _t1_body(a_ref, b_ref, out_ref):
    z = jnp.concatenate([a_ref[...], b_ref[...]], axis=0)
    out_ref[...] = z.T


def _t1(table_t):
    return pl.pallas_call(
        _t1_body,
        grid=(_T1_GRID,),
        in_specs=[
            pl.BlockSpec((HIDDEN_DIM, _T1_W), lambda g: (0, 2 * g)),
            pl.BlockSpec(
                (HIDDEN_DIM, _T1_W),
                lambda g: (0, jnp.minimum(2 * g + 1, _T1_LASTBLK))),
        ],
        out_specs=pl.BlockSpec((_T1_W, 2 * HIDDEN_DIM), lambda g: (g, 0)),
        out_shape=jax.ShapeDtypeStruct(
            (N_PAD // 2, 2 * HIDDEN_DIM), jnp.float32),
    )(table_t, table_t)


_T2_BLK = 256


def _t2_body(in_ref, out_ref):
    x = in_ref[...]
    x = x.reshape(_T2_BLK, 25, 128)
    for q in range(25):
        out_ref[q * 128:(q + 1) * 128, :] = x[:, q, :].T


def _t2(lin):
    return pl.pallas_call(
        _t2_body,
        grid=(BATCH // _T2_BLK,),
        in_specs=[pl.BlockSpec((25 * _T2_BLK, 128), lambda g: (g, 0))],
        out_specs=pl.BlockSpec((HIST * HIDDEN_DIM, _T2_BLK), lambda g: (0, g)),
        out_shape=jax.ShapeDtypeStruct((HIST * HIDDEN_DIM, BATCH), jnp.float32),
    )(lin)


def _make_gather():
    mesh = plsc.VectorSubcoreMesh(core_axis_name="c", subcore_axis_name="s")

    @functools.partial(
        pl.kernel,
        mesh=mesh,
        out_type=jax.ShapeDtypeStruct((TOTAL, HIDDEN_DIM), jnp.float32),
        compiler_params=pltpu.CompilerParams(use_tc_tiling_on_sc=False),
        scratch_types=[
            pltpu.VMEM((NCHUNK, CH), jnp.int32),
            pltpu.VMEM((GROW, HIDDEN_DIM), jnp.float32),
            pltpu.VMEM((GROW, HIDDEN_DIM), jnp.float32),
            pltpu.SemaphoreType.DMA,
            pltpu.SemaphoreType.DMA,
            pltpu.SemaphoreType.DMA,
            pltpu.SemaphoreType.DMA,
        ],
    )
    def gather_kernel(idx_hbm, table_hbm, out_hbm, idx_v, bank0, bank1,
                      gsem0, gsem1, wsem0, wsem1):
        wid = lax.axis_index("c") * NS + lax.axis_index("s")
        base = wid * PER_W
        banks = (bank0, bank1)
        gsems = (gsem0, gsem1)
        wsems = (wsem0, wsem1)

        pltpu.sync_copy(idx_hbm.at[wid], idx_v)

        def gather_desc(g, k, b):
            return pltpu.make_async_copy(
                table_hbm.at[idx_v.at[g * CPG + k]],
                banks[b].at[pl.ds(k * CH, CH)],
                gsems[b],
            )

        def write_desc(g, b):
            return pltpu.make_async_copy(
                banks[b],
                out_hbm.at[pl.ds(base + g * GROW, GROW)],
                wsems[b],
            )

        def start_gathers(g, b):
            for k in range(CPG):
                gather_desc(g, k, b).start()

        def wait_gathers(g, b):
            for k in range(CPG):
                gather_desc(g, k, b).wait()

        start_gathers(0, 0)
        start_gathers(1, 1)
        wait_gathers(0, 0)
        write_desc(0, 0).start()

        def do_group(g, b):
            write_desc(g - 2, b).wait()
            start_gathers(g, b)
            wait_gathers(g - 1, 1 - b)
            write_desc(g - 1, 1 - b).start()

        def body(t, carry):
            do_group(2 * t, 0)
            do_group(2 * t + 1, 1)
            return carry

        lax.fori_loop(1, NGRP // 2, body, 0)

        wait_gathers(NGRP - 1, 1)
        write_desc(NGRP - 1, 1).start()
        write_desc(NGRP - 2, 0).wait()
        write_desc(NGRP - 1, 1).wait()

    return gather_kernel


_gather = _make_gather()


def kernel(entity_ids, table):
    i = jnp.asarray(entity_ids, jnp.int32)
    r = (i & ~(_T1_SPAN - 1)) | ((i & (_T1_W - 1)) << 1) | (
        (i >> _T1_SHIFT) & 1)
    ids = r.reshape(NW, NCHUNK, CH)
    table_rm = _t1(table.T).reshape(N_PAD, HIDDEN_DIM)
    out_lin = _gather(ids, table_rm)
    o = _t2(out_lin.reshape(TOTAL // 2, 2 * HIDDEN_DIM))
    return o.reshape(HIST, HIDDEN_DIM, BATCH).transpose(2, 0, 1)


# speedup vs baseline: 3.8708x; 1.0390x over previous
import functools

import jax
import jax.numpy as jnp
from jax import lax
from jax.experimental import pallas as pl
from jax.experimental.pallas import tpu as pltpu
from jax.experimental.pallas import tpu_sc as plsc

NUM_ENTITIES = 1000000
HIDDEN_DIM = 64
BATCH = 16384
HIST = 50

NC = 2
NS = 16
NW = NC * NS
CH = 128
CPG = 5
GROW = CH * CPG

TOTAL = BATCH * HIST
HALF = TOTAL // 2
PER_W = HALF // NW
NGRP = PER_W // GROW
NCHUNK = PER_W // CH


_T1_W = 2048
_T1_SPAN = 2 * _T1_W
_T1_SHIFT = _T1_W.bit_length() - 1
_T1_GRID = (NUM_ENTITIES + _T1_SPAN - 1) // _T1_SPAN
_T1_LASTBLK = (NUM_ENTITIES - 1) // _T1_W
N_PAD = _T1_GRID * _T1_SPAN


def _t1_body(a_ref, b_ref, out_ref):
    z = jnp.concatenate([a_ref[...], b_ref[...]], axis=0)
    out_ref[...] = z.T


def _t1(table_t):
    return pl.pallas_call(
        _t1_body,
        grid=(_T1_GRID,),
        in_specs=[
            pl.BlockSpec((HIDDEN_DIM, _T1_W), lambda g: (0, 2 * g)),
            pl.BlockSpec(
                (HIDDEN_DIM, _T1_W),
                lambda g: (0, jnp.minimum(2 * g + 1, _T1_LASTBLK))),
        ],
        out_specs=pl.BlockSpec((_T1_W, 2 * HIDDEN_DIM), lambda g: (g, 0)),
        out_shape=jax.ShapeDtypeStruct(
            (N_PAD // 2, 2 * HIDDEN_DIM), jnp.float32),
    )(table_t, table_t)


_T2_BLK = 256
_T2_HGRID = BATCH // 2 // _T2_BLK


def _t2a_body(in_ref, out_ref):
    x = in_ref[...]
    x = x.reshape(_T2_BLK, 25, 128)
    for q in range(25):
        out_ref[q * 128:(q + 1) * 128, :] = x[:, q, :].T


def _t2b_body(in_ref, alias_ref, out_ref):
    del alias_ref
    _t2a_body(in_ref, out_ref)


def _t2a(lin_a):
    return pl.pallas_call(
        _t2a_body,
        grid=(_T2_HGRID,),
        in_specs=[pl.BlockSpec((25 * _T2_BLK, 128), lambda g: (g, 0))],
        out_specs=pl.BlockSpec((HIST * HIDDEN_DIM, _T2_BLK), lambda g: (0, g)),
        out_shape=jax.ShapeDtypeStruct((HIST * HIDDEN_DIM, BATCH), jnp.float32),
    )(lin_a)


def _t2b(lin_b, o_partial):
    return pl.pallas_call(
        _t2b_body,
        grid=(_T2_HGRID,),
        in_specs=[
            pl.BlockSpec((25 * _T2_BLK, 128), lambda g: (g, 0)),
            pl.BlockSpec(memory_space=pl.ANY),
        ],
        out_specs=pl.BlockSpec(
            (HIST * HIDDEN_DIM, _T2_BLK), lambda g: (0, g + _T2_HGRID)),
        out_shape=jax.ShapeDtypeStruct((HIST * HIDDEN_DIM, BATCH), jnp.float32),
        input_output_aliases={1: 0},
    )(lin_b, o_partial)


def _make_gather():
    mesh = plsc.VectorSubcoreMesh(core_axis_name="c", subcore_axis_name="s")

    @functools.partial(
        pl.kernel,
        mesh=mesh,
        out_type=jax.ShapeDtypeStruct((HALF, HIDDEN_DIM), jnp.float32),
        compiler_params=pltpu.CompilerParams(use_tc_tiling_on_sc=False),
        scratch_types=[
            pltpu.VMEM((NCHUNK, CH), jnp.int32),
            pltpu.VMEM((GROW, HIDDEN_DIM), jnp.float32),
            pltpu.VMEM((GROW, HIDDEN_DIM), jnp.float32),
            pltpu.SemaphoreType.DMA,
            pltpu.SemaphoreType.DMA,
            pltpu.SemaphoreType.DMA,
            pltpu.SemaphoreType.DMA,
        ],
    )
    def gather_kernel(idx_hbm, table_hbm, out_hbm, idx_v, bank0, bank1,
                      gsem0, gsem1, wsem0, wsem1):
        wid = lax.axis_index("c") * NS + lax.axis_index("s")
        base = wid * PER_W
        banks = (bank0, bank1)
        gsems = (gsem0, gsem1)
        wsems = (wsem0, wsem1)

        pltpu.sync_copy(idx_hbm.at[wid], idx_v)

        def gather_desc(g, k, b):
            return pltpu.make_async_copy(
                table_hbm.at[idx_v.at[g * CPG + k]],
                banks[b].at[pl.ds(k * CH, CH)],
                gsems[b],
            )

        def write_desc(g, b):
            return pltpu.make_async_copy(
                banks[b],
                out_hbm.at[pl.ds(base + g * GROW, GROW)],
                wsems[b],
            )

        def start_gathers(g, b):
            for k in range(CPG):
                gather_desc(g, k, b).start()

        def wait_gathers(g, b):
            for k in range(CPG):
                gather_desc(g, k, b).wait()

        start_gathers(0, 0)
        start_gathers(1, 1)
        wait_gathers(0, 0)
        write_desc(0, 0).start()

        def do_group(g, b):
            write_desc(g - 2, b).wait()
            start_gathers(g, b)
            wait_gathers(g - 1, 1 - b)
            write_desc(g - 1, 1 - b).start()

        def body(t, carry):
            do_group(2 * t, 0)
            do_group(2 * t + 1, 1)
            return carry

        lax.fori_loop(1, NGRP // 2, body, 0)

        wait_gathers(NGRP - 1, 1)
        write_desc(NGRP - 1, 1).start()
        write_desc(NGRP - 2, 0).wait()
        write_desc(NGRP - 1, 1).wait()

    return gather_kernel


_gather = _make_gather()


def kernel(entity_ids, table):
    i = jnp.asarray(entity_ids, jnp.int32)
    r = (i & ~(_T1_SPAN - 1)) | ((i & (_T1_W - 1)) << 1) | (
        (i >> _T1_SHIFT) & 1)
    ids_a = r[:BATCH // 2].reshape(NW, NCHUNK, CH)
    ids_b = r[BATCH // 2:].reshape(NW, NCHUNK, CH)
    table_rm = _t1(table.T).reshape(N_PAD, HIDDEN_DIM)
    lin_a = _gather(ids_a, table_rm)
    lin_b = _gather(ids_b, table_rm)
    o = _t2a(lin_a.reshape(HALF // 2, 2 * HIDDEN_DIM))
    o = _t2b(lin_b.reshape(HALF // 2, 2 * HIDDEN_DIM), o)
    return o.reshape(HIST, HIDDEN_DIM, BATCH).transpose(2, 0, 1)


# speedup vs baseline: 3.8949x; 1.0062x over previous
import functools

import jax
import jax.numpy as jnp
from jax import lax
from jax.experimental import pallas as pl
from jax.experimental.pallas import tpu as pltpu
from jax.experimental.pallas import tpu_sc as plsc

NUM_ENTITIES = 1000000
HIDDEN_DIM = 64
BATCH = 16384
HIST = 50

NC = 2
NS = 16
NW = NC * NS
CH = 128
CPG = 5
GROW = CH * CPG

TOTAL = BATCH * HIST
HALF = TOTAL // 2
PER_W = HALF // NW
NGRP = PER_W // GROW
NCHUNK = PER_W // CH


_T1_W = 2048
_T1_SPAN = 2 * _T1_W
_T1_SHIFT = _T1_W.bit_length() - 1
_T1_GRID = (NUM_ENTITIES + _T1_SPAN - 1) // _T1_SPAN
_T1_LASTBLK = (NUM_ENTITIES - 1) // _T1_W
N_PAD = _T1_GRID * _T1_SPAN


def _t1_body(a_ref, b_ref, out_ref):
    z = jnp.concatenate([a_ref[...], b_ref[...]], axis=0)
    out_ref[...] = z.T


def _t1(table_t):
    return pl.pallas_call(
        _t1_body,
        grid=(_T1_GRID,),
        in_specs=[
            pl.BlockSpec((HIDDEN_DIM, _T1_W), lambda g: (0, 2 * g)),
            pl.BlockSpec(
                (HIDDEN_DIM, _T1_W),
                lambda g: (0, jnp.minimum(2 * g + 1, _T1_LASTBLK))),
        ],
        out_specs=pl.BlockSpec((_T1_W, 2 * HIDDEN_DIM), lambda g: (g, 0)),
        out_shape=jax.ShapeDtypeStruct(
            (N_PAD // 2, 2 * HIDDEN_DIM), jnp.float32),
    )(table_t, table_t)


_T2_BLK = 256


def _t2_first_body(in_ref, out_ref):
    x = in_ref[...]
    x = x.reshape(_T2_BLK, 25, 128)
    for q in range(25):
        out_ref[q * 128:(q + 1) * 128, :] = x[:, q, :].T


def _t2_next_body(in_ref, alias_ref, out_ref):
    del alias_ref
    _t2_first_body(in_ref, out_ref)


def _t2(lin, nb, b_off, o_prev=None):
    grid = nb // _T2_BLK
    off = b_off // _T2_BLK
    out_shape = jax.ShapeDtypeStruct((HIST * HIDDEN_DIM, BATCH), jnp.float32)
    in_spec = pl.BlockSpec((25 * _T2_BLK, 128), lambda g: (g, 0))
    out_spec = pl.BlockSpec(
        (HIST * HIDDEN_DIM, _T2_BLK), lambda g: (0, g + off))
    if o_prev is None:
        return pl.pallas_call(
            _t2_first_body, grid=(grid,), in_specs=[in_spec],
            out_specs=out_spec, out_shape=out_shape)(lin)
    return pl.pallas_call(
        _t2_next_body, grid=(grid,),
        in_specs=[in_spec, pl.BlockSpec(memory_space=pl.ANY)],
        out_specs=out_spec, out_shape=out_shape,
        input_output_aliases={1: 0})(lin, o_prev)


def _make_gather(ngrp):
    per_w = ngrp * GROW
    nchunk = ngrp * CPG
    mesh = plsc.VectorSubcoreMesh(core_axis_name="c", subcore_axis_name="s")

    @functools.partial(
        pl.kernel,
        mesh=mesh,
        out_type=jax.ShapeDtypeStruct((NW * per_w, HIDDEN_DIM), jnp.float32),
        compiler_params=pltpu.CompilerParams(use_tc_tiling_on_sc=False),
        scratch_types=[
            pltpu.VMEM((nchunk, CH), jnp.int32),
            pltpu.VMEM((GROW, HIDDEN_DIM), jnp.float32),
            pltpu.VMEM((GROW, HIDDEN_DIM), jnp.float32),
            pltpu.SemaphoreType.DMA,
            pltpu.SemaphoreType.DMA,
            pltpu.SemaphoreType.DMA,
            pltpu.SemaphoreType.DMA,
        ],
    )
    def gather_kernel(idx_hbm, table_hbm, out_hbm, idx_v, bank0, bank1,
                      gsem0, gsem1, wsem0, wsem1):
        wid = lax.axis_index("c") * NS + lax.axis_index("s")
        base = wid * per_w
        banks = (bank0, bank1)
        gsems = (gsem0, gsem1)
        wsems = (wsem0, wsem1)

        pltpu.sync_copy(idx_hbm.at[wid], idx_v)

        def gather_desc(g, k, b):
            return pltpu.make_async_copy(
                table_hbm.at[idx_v.at[g * CPG + k]],
                banks[b].at[pl.ds(k * CH, CH)],
                gsems[b],
            )

        def write_desc(g, b):
            return pltpu.make_async_copy(
                banks[b],
                out_hbm.at[pl.ds(base + g * GROW, GROW)],
                wsems[b],
            )

        def start_gathers(g, b):
            for k in range(CPG):
                gather_desc(g, k, b).start()

        def wait_gathers(g, b):
            for k in range(CPG):
                gather_desc(g, k, b).wait()

        def do_group(g, b):
            write_desc(g - 2, b).wait()
            start_gathers(g, b)
            wait_gathers(g - 1, 1 - b)
            write_desc(g - 1, 1 - b).start()

        start_gathers(0, 0)
        start_gathers(1, 1)
        wait_gathers(0, 0)
        write_desc(0, 0).start()

        def body(t, carry):
            do_group(2 * t, 0)
            do_group(2 * t + 1, 1)
            return carry

        lax.fori_loop(1, ngrp // 2, body, 0)
        if ngrp % 2:
            do_group(ngrp - 1, 0)

        b_last = (ngrp - 1) % 2
        wait_gathers(ngrp - 1, b_last)
        write_desc(ngrp - 1, b_last).start()
        write_desc(ngrp - 2, 1 - b_last).wait()
        write_desc(ngrp - 1, b_last).wait()

    return gather_kernel


_SPLITS = (2048, 6144, 8192)
_GATHERS = {nb: _make_gather(nb * HIST // NW // GROW) for nb in set(_SPLITS)}


def kernel(entity_ids, table):
    i = jnp.asarray(entity_ids, jnp.int32)
    r = (i & ~(_T1_SPAN - 1)) | ((i & (_T1_W - 1)) << 1) | (
        (i >> _T1_SHIFT) & 1)
    table_rm = _t1(table.T).reshape(N_PAD, HIDDEN_DIM)
    lins = []
    b0 = 0
    for nb in _SPLITS:
        ids_k = r[b0:b0 + nb].reshape(NW, nb * HIST // NW // CH, CH)
        lins.append(_GATHERS[nb](ids_k, table_rm))
        b0 += nb
    o = None
    b0 = 0
    for nb, lin in zip(_SPLITS, lins):
        o = _t2(lin.reshape(nb * HIST // 2, 2 * HIDDEN_DIM), nb, b0, o)
        b0 += nb
    return o.reshape(HIST, HIDDEN_DIM, BATCH).transpose(2, 0, 1)
